# baseline (speedup 1.0000x reference)
import functools

import jax
import jax.numpy as jnp
from jax import lax
from jax.experimental import pallas as pl
from jax.experimental.pallas import tpu as pltpu
from jax.experimental.pallas import tpu_sc as plsc

B, S, D = 16, 2048, 512
FULL = 60000
HALF = FULL // 2
LANES = 16

TAIL_B = 10
HEAD_B = B - TAIL_B
TAIL_ROWS = TAIL_B * S
RPT = TAIL_ROWS // 32
CHUNK = 64
NCHUNKS = RPT // CHUNK


def _pool_body(h_ref, w_ref, b_ref, out_ref):
    hb = h_ref[...]
    w = w_ref[...][None, None, :]
    out_ref[...] = jnp.sum(hb * w, axis=-1) + b_ref[0]


def _pooled_tc(h, conv_w, conv_b, blk=32):
    n_head = HEAD_B * S // 128
    h3 = h.reshape(B * S // 128, 128, D)
    pooled = pl.pallas_call(
        _pool_body,
        grid=(n_head // blk,),
        in_specs=[
            pl.BlockSpec((blk, 128, D), lambda i: (i, 0, 0)),
            pl.BlockSpec((D,), lambda i: (0,)),
            pl.BlockSpec((1,), lambda i: (0,)),
        ],
        out_specs=pl.BlockSpec((blk, 128), lambda i: (i, 0)),
        out_shape=jax.ShapeDtypeStruct((n_head, 128), jnp.float32),
    )(h3, conv_w, conv_b)
    return pooled.reshape(HEAD_B * S)


_MESH = plsc.VectorSubcoreMesh(core_axis_name="c", subcore_axis_name="s")


@functools.partial(
    pl.kernel,
    mesh=_MESH,
    compiler_params=pltpu.CompilerParams(needs_layout_passes=False),
    out_type=jax.ShapeDtypeStruct((TAIL_ROWS,), jnp.float32),
    scratch_types=[
        pltpu.VMEM((CHUNK * D,), jnp.float32),
        pltpu.VMEM((CHUNK * D,), jnp.float32),
        pltpu.VMEM((D,), jnp.float32),
        pltpu.VMEM((LANES,), jnp.float32),
        pltpu.VMEM((RPT,), jnp.float32),
        pltpu.SemaphoreType.DMA,
        pltpu.SemaphoreType.DMA,
    ],
)
def _matvec_sc(h_hbm, w_hbm, b_hbm, out_hbm, buf0, buf1, w_v, b_v, out_v,
               s0, s1):
    wid = lax.axis_index("c") * 16 + lax.axis_index("s")
    row0 = HEAD_B * S + wid * RPT

    pltpu.sync_copy(w_hbm, w_v)
    pltpu.sync_copy(b_hbm, b_v)
    bias = b_v[...]
    iota = lax.iota(jnp.int32, LANES)

    bufs = (buf0, buf1)
    sems = (s0, s1)
    cps = {0: pltpu.async_copy(
        h_hbm.at[pl.ds(row0 * D, CHUNK * D)], buf0, s0)}
    for c in range(NCHUNKS):
        if c + 1 < NCHUNKS:
            nb = (c + 1) % 2
            cps[c + 1] = pltpu.async_copy(
                h_hbm.at[pl.ds((row0 + (c + 1) * CHUNK) * D, CHUNK * D)],
                bufs[nb], sems[nb])
        cps[c].wait()
        buf = bufs[c % 2]
        for g in range(CHUNK // LANES):
            idx0 = iota * D + (g * LANES) * D

            def dvstep(dv, carry, buf=buf):
                acc, idx = carry
                wv = w_v[pl.ds(dv * LANES, LANES)]
                for l in range(LANES):
                    hv = plsc.load_gather(buf, [idx])
                    acc = acc + hv * jnp.full((LANES,), wv[l], jnp.float32)
                    idx = idx + 1
                return (acc, idx)

            acc, _ = lax.fori_loop(
                0, D // LANES, dvstep,
                (jnp.zeros((LANES,), jnp.float32), idx0))
            out_v[pl.ds(c * CHUNK + g * LANES, LANES)] = acc + bias

    pltpu.sync_copy(out_v, out_hbm.at[pl.ds(wid * RPT, RPT)])


@functools.partial(
    pl.kernel,
    mesh=_MESH,
    compiler_params=pltpu.CompilerParams(needs_layout_passes=False),
    out_type=jax.ShapeDtypeStruct((B * FULL,), jnp.float32),
    scratch_types=[
        pltpu.VMEM((S,), jnp.int32),
        pltpu.VMEM((S,), jnp.float32),
        pltpu.VMEM((HALF,), jnp.float32),
        pltpu.SemaphoreType.DMA,
        pltpu.SemaphoreType.DMA,
    ],
)
def _scatter_sc(idx_hbm, val_hbm, out_hbm, idx_v, val_v, row_v, sem_i, sem_v):
    wid = lax.axis_index("c") * 16 + lax.axis_index("s")
    b = wid // 2
    lo = (wid % 2) * HALF

    cp_i = pltpu.async_copy(idx_hbm.at[pl.ds(b * S, S)], idx_v, sem_i)
    cp_v = pltpu.async_copy(val_hbm.at[pl.ds(b * S, S)], val_v, sem_v)

    zeros = jnp.zeros((LANES,), jnp.float32)

    def zero_body(j, _):
        base = j * (15 * LANES)
        for u in range(15):
            row_v[pl.ds(base + u * LANES, LANES)] = zeros
        return 0

    lax.fori_loop(0, HALF // (15 * LANES), zero_body, 0)

    cp_i.wait()
    cp_v.wait()

    def scat_body(i, _):
        base = i * (4 * LANES)
        for u in range(4):
            idx = idx_v[pl.ds(base + u * LANES, LANES)]
            val = val_v[pl.ds(base + u * LANES, LANES)]
            local = idx - lo
            mask = (local >= 0) & (local < HALF)
            plsc.store_scatter(row_v, [local], val, mask=mask)
        return 0

    lax.fori_loop(0, S // (4 * LANES), scat_body, 0)

    pltpu.sync_copy(row_v, out_hbm.at[pl.ds(b * FULL + lo, HALF)])


def kernel(h, gene_pos, conv_w, conv_b):
    pooled_head = _pooled_tc(h, conv_w, conv_b)
    b16 = jnp.broadcast_to(conv_b, (LANES,))
    pooled_tail = _matvec_sc(h.reshape(B * S * D), conv_w, b16)
    pooled = jnp.concatenate([pooled_head, pooled_tail])
    out = _scatter_sc(gene_pos.reshape(B * S), pooled)
    return out.reshape(B, FULL)


# speedup vs baseline: 1.2743x; 1.2743x over previous
import functools

import jax
import jax.numpy as jnp
from jax import lax
from jax.experimental import pallas as pl
from jax.experimental.pallas import tpu as pltpu
from jax.experimental.pallas import tpu_sc as plsc

B, S, D = 16, 2048, 512
FULL = 60000
HALF = FULL // 2
LANES = 16

TAIL_B = 10
HEAD_B = B - TAIL_B
TAIL_ROWS = TAIL_B * S
RPT = TAIL_ROWS // 32
CHUNK = 64
NCHUNKS = RPT // CHUNK


def _pool_body(h_ref, w_ref, b_ref, out_ref):
    hb = h_ref[...]
    w = w_ref[...][None, None, :]
    out_ref[...] = jnp.sum(hb * w, axis=-1) + b_ref[0]


def _pooled_tc(h, conv_w, conv_b, blk=32):
    n_head = HEAD_B * S // 128
    h3 = h.reshape(B * S // 128, 128, D)
    pooled = pl.pallas_call(
        _pool_body,
        grid=(n_head // blk,),
        in_specs=[
            pl.BlockSpec((blk, 128, D), lambda i: (i, 0, 0)),
            pl.BlockSpec((D,), lambda i: (0,)),
            pl.BlockSpec((1,), lambda i: (0,)),
        ],
        out_specs=pl.BlockSpec((blk, 128), lambda i: (i, 0)),
        out_shape=jax.ShapeDtypeStruct((n_head, 128), jnp.float32),
    )(h3, conv_w, conv_b)
    return pooled.reshape(HEAD_B * S)


_MESH = plsc.VectorSubcoreMesh(core_axis_name="c", subcore_axis_name="s")


@functools.partial(
    pl.kernel,
    mesh=_MESH,
    compiler_params=pltpu.CompilerParams(needs_layout_passes=False),
    out_type=jax.ShapeDtypeStruct((TAIL_ROWS,), jnp.float32),
    scratch_types=[
        pltpu.VMEM((CHUNK, D), jnp.float32),
        pltpu.VMEM((CHUNK, D), jnp.float32),
        pltpu.VMEM((D,), jnp.float32),
        pltpu.VMEM((LANES,), jnp.float32),
        pltpu.VMEM((RPT,), jnp.float32),
        pltpu.SemaphoreType.DMA,
        pltpu.SemaphoreType.DMA,
    ],
)
def _matvec_sc(h_hbm, w_hbm, b_hbm, out_hbm, buf0, buf1, w_v, b_v, out_v,
               s0, s1):
    wid = lax.axis_index("c") * 16 + lax.axis_index("s")
    row0 = HEAD_B * S + wid * RPT

    pltpu.sync_copy(w_hbm, w_v)
    pltpu.sync_copy(b_hbm, b_v)
    bias = b_v[...]
    iota = lax.iota(jnp.int32, LANES)

    bufs = (buf0, buf1)
    sems = (s0, s1)
    cps = {0: pltpu.async_copy(
        h_hbm.at[pl.ds(row0, CHUNK)], buf0, s0)}
    for c in range(NCHUNKS):
        if c + 1 < NCHUNKS:
            nb = (c + 1) % 2
            cps[c + 1] = pltpu.async_copy(
                h_hbm.at[pl.ds(row0 + (c + 1) * CHUNK, CHUNK)],
                bufs[nb], sems[nb])
        cps[c].wait()
        buf = bufs[c % 2]
        for g in range(CHUNK // LANES):
            rows_g = iota + (g * LANES)

            def dvstep(dv, carry, buf=buf, rows_g=rows_g):
                acc, dvec = carry
                wv = w_v[pl.ds(dv * LANES, LANES)]
                for l in range(LANES):
                    hv = plsc.load_gather(buf, [rows_g, dvec])
                    acc = acc + hv * jnp.full((LANES,), wv[l], jnp.float32)
                    dvec = dvec + 1
                return (acc, dvec)

            acc, _ = lax.fori_loop(
                0, D // LANES, dvstep,
                (jnp.zeros((LANES,), jnp.float32),
                 jnp.zeros((LANES,), jnp.int32)))
            out_v[pl.ds(c * CHUNK + g * LANES, LANES)] = acc + bias

    pltpu.sync_copy(out_v, out_hbm.at[pl.ds(wid * RPT, RPT)])


@functools.partial(
    pl.kernel,
    mesh=_MESH,
    compiler_params=pltpu.CompilerParams(needs_layout_passes=False),
    out_type=jax.ShapeDtypeStruct((B * FULL,), jnp.float32),
    scratch_types=[
        pltpu.VMEM((S,), jnp.int32),
        pltpu.VMEM((S,), jnp.float32),
        pltpu.VMEM((HALF,), jnp.float32),
        pltpu.SemaphoreType.DMA,
        pltpu.SemaphoreType.DMA,
    ],
)
def _scatter_sc(idx_hbm, val_hbm, out_hbm, idx_v, val_v, row_v, sem_i, sem_v):
    wid = lax.axis_index("c") * 16 + lax.axis_index("s")
    b = wid // 2
    lo = (wid % 2) * HALF

    cp_i = pltpu.async_copy(idx_hbm.at[pl.ds(b * S, S)], idx_v, sem_i)
    cp_v = pltpu.async_copy(val_hbm.at[pl.ds(b * S, S)], val_v, sem_v)

    zeros = jnp.zeros((LANES,), jnp.float32)

    def zero_body(j, _):
        base = j * (15 * LANES)
        for u in range(15):
            row_v[pl.ds(base + u * LANES, LANES)] = zeros
        return 0

    lax.fori_loop(0, HALF // (15 * LANES), zero_body, 0)

    cp_i.wait()
    cp_v.wait()

    def scat_body(i, _):
        base = i * (4 * LANES)
        for u in range(4):
            idx = idx_v[pl.ds(base + u * LANES, LANES)]
            val = val_v[pl.ds(base + u * LANES, LANES)]
            local = idx - lo
            mask = (local >= 0) & (local < HALF)
            plsc.store_scatter(row_v, [local], val, mask=mask)
        return 0

    lax.fori_loop(0, S // (4 * LANES), scat_body, 0)

    pltpu.sync_copy(row_v, out_hbm.at[pl.ds(b * FULL + lo, HALF)])


def kernel(h, gene_pos, conv_w, conv_b):
    pooled_head = _pooled_tc(h, conv_w, conv_b)
    b16 = jnp.broadcast_to(conv_b, (LANES,))
    pooled_tail = _matvec_sc(h.reshape(B * S, D), conv_w, b16)
    pooled = jnp.concatenate([pooled_head, pooled_tail])
    out = _scatter_sc(gene_pos.reshape(B * S), pooled)
    return out.reshape(B, FULL)


# speedup vs baseline: 4.5347x; 3.5585x over previous
import functools

import jax
import jax.numpy as jnp
from jax import lax
from jax.experimental import pallas as pl
from jax.experimental.pallas import tpu as pltpu
from jax.experimental.pallas import tpu_sc as plsc

B, S, D = 16, 2048, 512
FULL = 60000
HALF = FULL // 2
LANES = 16


def _pool_body(h_ref, w_ref, b_ref, out_ref):
    hb = h_ref[...]
    w = w_ref[...][None, None, :]
    out_ref[...] = jnp.sum(hb * w, axis=-1) + b_ref[0]


def _pooled_tc(h, conv_w, conv_b, blk=32):
    n = B * S // 128
    h3 = h.reshape(n, 128, D)
    pooled = pl.pallas_call(
        _pool_body,
        grid=(n // blk,),
        in_specs=[
            pl.BlockSpec((blk, 128, D), lambda i: (i, 0, 0)),
            pl.BlockSpec((D,), lambda i: (0,)),
            pl.BlockSpec((1,), lambda i: (0,)),
        ],
        out_specs=pl.BlockSpec((blk, 128), lambda i: (i, 0)),
        out_shape=jax.ShapeDtypeStruct((n, 128), jnp.float32),
    )(h3, conv_w, conv_b)
    return pooled.reshape(B, S)


_MESH = plsc.VectorSubcoreMesh(core_axis_name="c", subcore_axis_name="s")


@functools.partial(
    pl.kernel,
    mesh=_MESH,
    compiler_params=pltpu.CompilerParams(needs_layout_passes=False),
    out_type=jax.ShapeDtypeStruct((B * FULL,), jnp.float32),
    scratch_types=[
        pltpu.VMEM((S,), jnp.int32),
        pltpu.VMEM((S,), jnp.float32),
        pltpu.VMEM((HALF,), jnp.float32),
        pltpu.SemaphoreType.DMA,
        pltpu.SemaphoreType.DMA,
    ],
)
def _scatter_sc(idx_hbm, val_hbm, out_hbm, idx_v, val_v, row_v, sem_i, sem_v):
    wid = lax.axis_index("c") * 16 + lax.axis_index("s")
    b = wid // 2
    lo = (wid % 2) * HALF

    cp_i = pltpu.async_copy(idx_hbm.at[pl.ds(b * S, S)], idx_v, sem_i)
    cp_v = pltpu.async_copy(val_hbm.at[pl.ds(b * S, S)], val_v, sem_v)

    zeros = jnp.zeros((LANES,), jnp.float32)

    def zero_body(j, _):
        base = j * (15 * LANES)
        for u in range(15):
            row_v[pl.ds(base + u * LANES, LANES)] = zeros
        return 0

    lax.fori_loop(0, HALF // (15 * LANES), zero_body, 0)

    cp_i.wait()
    cp_v.wait()

    def scat_body(i, _):
        base = i * (4 * LANES)
        for u in range(4):
            idx = idx_v[pl.ds(base + u * LANES, LANES)]
            val = val_v[pl.ds(base + u * LANES, LANES)]
            local = idx - lo
            mask = (local >= 0) & (local < HALF)
            plsc.store_scatter(row_v, [local], val, mask=mask)
        return 0

    lax.fori_loop(0, S // (4 * LANES), scat_body, 0)

    pltpu.sync_copy(row_v, out_hbm.at[pl.ds(b * FULL + lo, HALF)])


def kernel(h, gene_pos, conv_w, conv_b):
    pooled = _pooled_tc(h, conv_w, conv_b)
    out = _scatter_sc(gene_pos.reshape(B * S), pooled.reshape(B * S))
    return out.reshape(B, FULL)
